# initial kernel scaffold (unmeasured)
import jax
import jax.numpy as jnp
from jax import lax
from jax.experimental import pallas as pl
from jax.experimental.pallas import tpu as pltpu


def kernel(O, Wo):
    B, S, H, D = O.shape
    HD = H * D
    N = Wo.shape[1]
    s_half = S // 2

    O = O.reshape(B, S, HD).astype(jnp.bfloat16)
    Wo = Wo.astype(jnp.bfloat16)

    def body(o_ref, w_ref, out_ref, send_buf, recv_buf, send_sems, recv_sems):
        my_x = lax.axis_index("x")
        my_y = lax.axis_index("y")
        peer = (1 - my_x, my_y)

        barrier = pltpu.get_barrier_semaphore()
        pl.semaphore_signal(
            barrier, inc=1, device_id=peer,
            device_id_type=pl.DeviceIdType.MESH,
        )
        pl.semaphore_wait(barrier, 1)

        peer_s0 = (1 - my_x) * s_half
        my_s0 = my_x * s_half

        rdmas = []
        for b in range(B):
            o_b = o_ref[b, pl.ds(peer_s0, s_half), :]
            p = jnp.dot(o_b, w_ref[...], preferred_element_type=jnp.float32)
            send_buf[b, :, :] = p.astype(jnp.bfloat16)
            rdma = pltpu.make_async_remote_copy(
                src_ref=send_buf.at[b],
                dst_ref=recv_buf.at[b],
                send_sem=send_sems.at[b],
                recv_sem=recv_sems.at[b],
                device_id=peer,
                device_id_type=pl.DeviceIdType.MESH,
            )
            rdma.start()
            rdmas.append(rdma)

        for b in range(B):
            o_b = o_ref[b, pl.ds(my_s0, s_half), :]
            p = jnp.dot(o_b, w_ref[...], preferred_element_type=jnp.float32)
            rdmas[b].wait()
            out_ref[b, :, :] = p + recv_buf[b, :, :].astype(jnp.float32)

    return pl.pallas_call(
        body,
        out_shape=jax.ShapeDtypeStruct((B, s_half, N), jnp.float32),
        in_specs=[
            pl.BlockSpec(memory_space=pltpu.VMEM),
            pl.BlockSpec(memory_space=pltpu.VMEM),
        ],
        out_specs=pl.BlockSpec(memory_space=pltpu.VMEM),
        scratch_shapes=[
            pltpu.VMEM((B, s_half, N), jnp.bfloat16),
            pltpu.VMEM((B, s_half, N), jnp.bfloat16),
            pltpu.SemaphoreType.DMA((B,)),
            pltpu.SemaphoreType.DMA((B,)),
        ],
        compiler_params=pltpu.CompilerParams(collective_id=0),
    )(O, Wo)


# baseline (device time: 277653 ns/iter reference)
import jax
import jax.numpy as jnp
from jax import lax
from jax.experimental import pallas as pl
from jax.experimental.pallas import tpu as pltpu


def kernel(O, Wo):
    B, S, H, D = O.shape
    HD = H * D
    N = Wo.shape[1]
    s_half = S // 2

    O = O.reshape(B, S, HD).astype(jnp.bfloat16)
    Wo = Wo.astype(jnp.bfloat16)

    def body(o_hbm, w_ref, out_hbm, recv_hbm,
             send_buf, o_tile, recv_tile, out_tile,
             send_sems, recv_sems, load_sem, stage_sem, store_sem):
        my_x = lax.axis_index("x")
        my_y = lax.axis_index("y")
        peer = (1 - my_x, my_y)

        barrier = pltpu.get_barrier_semaphore()
        pl.semaphore_signal(
            barrier, inc=1, device_id=peer,
            device_id_type=pl.DeviceIdType.MESH,
        )
        pl.semaphore_wait(barrier, 1)

        peer_s0 = (1 - my_x) * s_half
        my_s0 = my_x * s_half

        def load_o(b, s0):
            cp = pltpu.make_async_copy(
                o_hbm.at[b, pl.ds(s0, s_half), :], o_tile, load_sem,
            )
            cp.start()
            cp.wait()

        rdmas = []
        for b in range(B):
            load_o(b, peer_s0)
            p = jnp.dot(o_tile[...], w_ref[...],
                        preferred_element_type=jnp.float32)
            send_buf[b, :, :] = p.astype(jnp.bfloat16)
            rdma = pltpu.make_async_remote_copy(
                src_ref=send_buf.at[b],
                dst_ref=recv_hbm.at[b],
                send_sem=send_sems.at[b],
                recv_sem=recv_sems.at[b],
                device_id=peer,
                device_id_type=pl.DeviceIdType.MESH,
            )
            rdma.start()
            rdmas.append(rdma)

        for b in range(B):
            load_o(b, my_s0)
            p = jnp.dot(o_tile[...], w_ref[...],
                        preferred_element_type=jnp.float32)
            rdmas[b].wait()
            stage = pltpu.make_async_copy(
                recv_hbm.at[b], recv_tile, stage_sem,
            )
            stage.start()
            stage.wait()
            out_tile[...] = (p + recv_tile[...].astype(jnp.float32)
                             ).astype(jnp.bfloat16)
            store = pltpu.make_async_copy(
                out_tile, out_hbm.at[b], store_sem,
            )
            store.start()
            store.wait()

    out, _ = pl.pallas_call(
        body,
        out_shape=[
            jax.ShapeDtypeStruct((B, s_half, N), jnp.bfloat16),
            jax.ShapeDtypeStruct((B, s_half, N), jnp.bfloat16),
        ],
        in_specs=[
            pl.BlockSpec(memory_space=pl.ANY),
            pl.BlockSpec(memory_space=pltpu.VMEM),
        ],
        out_specs=[
            pl.BlockSpec(memory_space=pl.ANY),
            pl.BlockSpec(memory_space=pl.ANY),
        ],
        scratch_shapes=[
            pltpu.VMEM((B, s_half, N), jnp.bfloat16),
            pltpu.VMEM((s_half, HD), jnp.bfloat16),
            pltpu.VMEM((s_half, N), jnp.bfloat16),
            pltpu.VMEM((s_half, N), jnp.bfloat16),
            pltpu.SemaphoreType.DMA((B,)),
            pltpu.SemaphoreType.DMA((B,)),
            pltpu.SemaphoreType.DMA,
            pltpu.SemaphoreType.DMA,
            pltpu.SemaphoreType.DMA,
        ],
        compiler_params=pltpu.CompilerParams(
            collective_id=0,
            vmem_limit_bytes=100 * 1024 * 1024,
        ),
    )(O, Wo)
    return out.astype(jnp.float32)


# device time: 188059 ns/iter; 1.4764x vs baseline; 1.4764x over previous
import jax
import jax.numpy as jnp
from jax import lax
from jax.experimental import pallas as pl
from jax.experimental.pallas import tpu as pltpu


def kernel(O, Wo):
    B, S, H, D = O.shape
    HD = H * D
    N = Wo.shape[1]
    s_half = S // 2
    n_half = N // 2

    O = O.reshape(B, S, HD)

    def body(o_hbm, w_hbm, out_hbm,
             w_f32, w_buf, o_tile, send_buf, rx_buf,
             sx_sems, rx_sems, sy_sems, ry_sems,
             w_sem, load_sem, store_sem):
        my_x = lax.axis_index("x")
        my_y = lax.axis_index("y")
        x_peer = (1 - my_x, my_y)
        y_peer = (my_x, 1 - my_y)

        barrier = pltpu.get_barrier_semaphore()
        for nbr in (x_peer, y_peer):
            pl.semaphore_signal(
                barrier, inc=1, device_id=nbr,
                device_id_type=pl.DeviceIdType.MESH,
            )
        pl.semaphore_wait(barrier, 2)

        peer_s0 = (1 - my_x) * s_half
        my_s0 = my_x * s_half
        col0 = my_y * n_half

        wcp = pltpu.make_async_copy(
            w_hbm.at[:, pl.ds(col0, n_half)], w_f32, w_sem,
        )
        wcp.start()
        wcp.wait()
        w_buf[...] = w_f32[...].astype(jnp.bfloat16)

        def load_o(b, s0):
            cp = pltpu.make_async_copy(
                o_hbm.at[b, pl.ds(s0, s_half), :], o_tile, load_sem,
            )
            cp.start()
            cp.wait()

        x_rdmas = []
        for b in range(B):
            load_o(b, peer_s0)
            p = jnp.dot(o_tile[...].astype(jnp.bfloat16), w_buf[...],
                        preferred_element_type=jnp.float32)
            send_buf[b, :, :] = p.astype(jnp.bfloat16)
            rdma = pltpu.make_async_remote_copy(
                src_ref=send_buf.at[b],
                dst_ref=rx_buf.at[b],
                send_sem=sx_sems.at[b],
                recv_sem=rx_sems.at[b],
                device_id=x_peer,
                device_id_type=pl.DeviceIdType.MESH,
            )
            rdma.start()
            x_rdmas.append(rdma)

        y_rdmas = []
        for b in range(B):
            load_o(b, my_s0)
            p = jnp.dot(o_tile[...].astype(jnp.bfloat16), w_buf[...],
                        preferred_element_type=jnp.float32)
            x_rdmas[b].wait()
            final = (p + rx_buf[b, :, :].astype(jnp.float32)
                     ).astype(jnp.bfloat16)
            send_buf[b, :, :] = final
            st = pltpu.make_async_copy(
                send_buf.at[b], out_hbm.at[b, :, pl.ds(col0, n_half)],
                store_sem,
            )
            st.start()
            st.wait()
            rdma = pltpu.make_async_remote_copy(
                src_ref=send_buf.at[b],
                dst_ref=out_hbm.at[b, :, pl.ds(col0, n_half)],
                send_sem=sy_sems.at[b],
                recv_sem=ry_sems.at[b],
                device_id=y_peer,
                device_id_type=pl.DeviceIdType.MESH,
            )
            rdma.start()
            y_rdmas.append(rdma)

        for b in range(B):
            y_rdmas[b].wait()

    out = pl.pallas_call(
        body,
        out_shape=jax.ShapeDtypeStruct((B, s_half, N), jnp.bfloat16),
        in_specs=[
            pl.BlockSpec(memory_space=pl.ANY),
            pl.BlockSpec(memory_space=pl.ANY),
        ],
        out_specs=pl.BlockSpec(memory_space=pl.ANY),
        scratch_shapes=[
            pltpu.VMEM((HD, n_half), jnp.float32),
            pltpu.VMEM((HD, n_half), jnp.bfloat16),
            pltpu.VMEM((s_half, HD), jnp.float32),
            pltpu.VMEM((B, s_half, n_half), jnp.bfloat16),
            pltpu.VMEM((B, s_half, n_half), jnp.bfloat16),
            pltpu.SemaphoreType.DMA((B,)),
            pltpu.SemaphoreType.DMA((B,)),
            pltpu.SemaphoreType.DMA((B,)),
            pltpu.SemaphoreType.DMA((B,)),
            pltpu.SemaphoreType.DMA,
            pltpu.SemaphoreType.DMA,
            pltpu.SemaphoreType.DMA,
        ],
        compiler_params=pltpu.CompilerParams(
            collective_id=0,
            vmem_limit_bytes=100 * 1024 * 1024,
        ),
    )(O, Wo)
    return out
